# Initial kernel scaffold; baseline (speedup 1.0000x reference)
import jax
import jax.numpy as jnp
from jax.experimental import pallas as pl


def kernel(x, tables):
    raise NotImplementedError("write your pallas kernel here")


# speedup vs baseline: 1.4562x; 1.4562x over previous
import functools

import jax
import jax.numpy as jnp
import numpy as np
from jax import lax
from jax.experimental import pallas as pl
from jax.experimental.pallas import tpu as pltpu
from jax.experimental.pallas import tpu_sc as plsc

_FIELD_DIMS = [1000] * 26
_NUM_FIELDS = len(_FIELD_DIMS)
_TOTAL = int(sum(_FIELD_DIMS))
_EMBED_DIM = 32
_BATCH = 4096
_ROW_STRIDE = _TOTAL + _FIELD_DIMS[0]

_NUM_CORES = 2
_NUM_SUBCORES = 16
_NUM_WORKERS = _NUM_CORES * _NUM_SUBCORES
_N = _BATCH * _NUM_FIELDS
_PER_W = _N // _NUM_WORKERS
_CHUNK = 128
_NCHUNK = _PER_W // _CHUNK
_LANES = 16


def _body(x_hbm, tab_hbm, out_hbm, idx_v, rows_v, gsem, osem):
    wid = lax.axis_index("s") * _NUM_CORES + lax.axis_index("c")
    base = wid * _PER_W

    pltpu.sync_copy(x_hbm.at[pl.ds(base, _PER_W)], idx_v)

    def add_off(j, _):
        s = j * _LANES
        pos = s + lax.iota(jnp.int32, _LANES)
        fid = pos % _NUM_FIELDS
        idx_v[pl.ds(s, _LANES)] = idx_v[pl.ds(s, _LANES)] + fid * _ROW_STRIDE
        return _

    lax.fori_loop(0, _PER_W // _LANES, add_off, None)

    gathers = []
    for j in range(_NCHUNK):
        gathers.append(pltpu.async_copy(
            tab_hbm.at[idx_v.at[pl.ds(j * _CHUNK, _CHUNK)]],
            rows_v.at[pl.ds(j * _CHUNK, _CHUNK)],
            gsem,
        ))
    outs = []
    for j in range(_NCHUNK):
        gathers[j].wait()
        outs.append(pltpu.async_copy(
            rows_v.at[pl.ds(j * _CHUNK, _CHUNK)],
            out_hbm.at[pl.ds(base + j * _CHUNK, _CHUNK)],
            osem,
        ))
    for o in outs:
        o.wait()


@jax.jit
def kernel(x, tables):
    tab_flat = tables.reshape(_NUM_FIELDS * _TOTAL, _EMBED_DIM)
    x_flat = x.reshape(_N)
    mesh = plsc.VectorSubcoreMesh(core_axis_name="c", subcore_axis_name="s")
    out = pl.kernel(
        _body,
        out_type=jax.ShapeDtypeStruct((_N, _EMBED_DIM), jnp.float32),
        mesh=mesh,
        scratch_types=[
            pltpu.VMEM((_PER_W,), jnp.int32),
            pltpu.VMEM((_PER_W, _EMBED_DIM), jnp.float32),
            pltpu.SemaphoreType.DMA,
            pltpu.SemaphoreType.DMA,
        ],
        compiler_params=pltpu.CompilerParams(use_tc_tiling_on_sc=False),
    )(x_flat, tab_flat)
    return out.reshape(_BATCH, _NUM_FIELDS, _EMBED_DIM)


# speedup vs baseline: 2.9517x; 2.0270x over previous
import functools

import jax
import jax.numpy as jnp
import numpy as np
from jax import lax
from jax.experimental import pallas as pl
from jax.experimental.pallas import tpu as pltpu
from jax.experimental.pallas import tpu_sc as plsc

_FIELD_DIMS = [1000] * 26
_NUM_FIELDS = len(_FIELD_DIMS)
_TOTAL = int(sum(_FIELD_DIMS))
_EMBED_DIM = 32
_BATCH = 4096
_HOT = _FIELD_DIMS[0]
_ROW_STRIDE = _HOT

_NUM_CORES = 2
_NUM_SUBCORES = 16
_NUM_WORKERS = _NUM_CORES * _NUM_SUBCORES
_N = _BATCH * _NUM_FIELDS
_PER_W = _N // _NUM_WORKERS
_CHUNK = 128
_NCHUNK = _PER_W // _CHUNK
_LANES = 16


def _body(x_hbm, tab_hbm, out_hbm, idx_v, rows_v, gsem, osem):
    wid = lax.axis_index("s") * _NUM_CORES + lax.axis_index("c")
    base = wid * _PER_W

    pltpu.sync_copy(x_hbm.at[pl.ds(base, _PER_W)], idx_v)

    def add_off(j, _):
        s = j * _LANES
        pos = s + lax.iota(jnp.int32, _LANES)
        fid = pos % _NUM_FIELDS
        idx_v[pl.ds(s, _LANES)] = idx_v[pl.ds(s, _LANES)] + fid * _ROW_STRIDE
        return _

    lax.fori_loop(0, _PER_W // _LANES, add_off, None)

    gathers = []
    for j in range(_NCHUNK):
        gathers.append(pltpu.async_copy(
            tab_hbm.at[idx_v.at[pl.ds(j * _CHUNK, _CHUNK)]],
            rows_v.at[pl.ds(j * _CHUNK, _CHUNK)],
            gsem,
        ))
    outs = []
    for j in range(_NCHUNK):
        gathers[j].wait()
        outs.append(pltpu.async_copy(
            rows_v.at[pl.ds(j * _CHUNK, _CHUNK)],
            out_hbm.at[pl.ds(base + j * _CHUNK, _CHUNK)],
            osem,
        ))
    for o in outs:
        o.wait()


@jax.jit
def kernel(x, tables):
    hot = jnp.stack([
        lax.slice_in_dim(tables[f], f * _HOT, (f + 1) * _HOT, axis=0)
        for f in range(_NUM_FIELDS)
    ])
    tab_flat = hot.reshape(_NUM_FIELDS * _HOT, _EMBED_DIM)
    x_flat = x.reshape(_N)
    mesh = plsc.VectorSubcoreMesh(core_axis_name="c", subcore_axis_name="s")
    out = pl.kernel(
        _body,
        out_type=jax.ShapeDtypeStruct((_N, _EMBED_DIM), jnp.float32),
        mesh=mesh,
        scratch_types=[
            pltpu.VMEM((_PER_W,), jnp.int32),
            pltpu.VMEM((_PER_W, _EMBED_DIM), jnp.float32),
            pltpu.SemaphoreType.DMA,
            pltpu.SemaphoreType.DMA,
        ],
        compiler_params=pltpu.CompilerParams(use_tc_tiling_on_sc=False),
    )(x_flat, tab_flat)
    return out.reshape(_BATCH, _NUM_FIELDS, _EMBED_DIM)
